# baseline (speedup 1.0000x reference)
import jax
import jax.numpy as jnp
from jax import lax
from jax.experimental import pallas as pl
from jax.experimental.pallas import tpu as pltpu
from jax.experimental.pallas import tpu_sc as plsc

VOCAB = 1000000
EMBED_DIM = 64
BATCH = 16384
LANES = 16
NUM_WORKERS = 32
B_PER_W = BATCH // NUM_WORKERS
GATHER_CHUNK = 128
N_CHUNKS = B_PER_W // GATHER_CHUNK
GROUPS = B_PER_W // LANES
P_ROWS = VOCAB // LANES

TC_CHUNK = 32768
SC_V = 262144
SC_PER_W = SC_V // NUM_WORKERS
SC_MV_CHUNK = 512
SC_MV_STEPS = SC_PER_W // SC_MV_CHUNK
TC_V = VOCAB - SC_V
TC_GRID = (TC_V + TC_CHUNK - 1) // TC_CHUNK
TC_BLK0 = SC_V // TC_CHUNK


def _matvec_body(t_ref, w_ref, b_ref, p_ref):
    p_ref[...] = jnp.sum(t_ref[...] * w_ref[...], axis=0) + b_ref[0, 0]


def _table_dot_w_tc(tphys, w, b11):
    return pl.pallas_call(
        _matvec_body,
        grid=(TC_GRID,),
        in_specs=[
            pl.BlockSpec((EMBED_DIM, TC_CHUNK), lambda i: (0, i + TC_BLK0)),
            pl.BlockSpec((EMBED_DIM, 1), lambda i: (0, 0)),
            pl.BlockSpec((1, 1), lambda i: (0, 0)),
        ],
        out_specs=pl.BlockSpec((TC_CHUNK,), lambda i: (i,)),
        out_shape=jax.ShapeDtypeStruct((TC_V,), jnp.float32),
    )(tphys, w, b11)


def _sc_mv_body(tp_hbm, w_hbm, bias_hbm, out_hbm, tv0, tv1, w_v, bias_v,
                out_v, sem0, sem1):
    wid = lax.axis_index("s") * 2 + lax.axis_index("c")
    base_v = pl.multiple_of(wid * SC_PER_W, 1024)

    pltpu.sync_copy(w_hbm, w_v)
    pltpu.sync_copy(bias_hbm, bias_v)
    w_vecs = [w_v[pl.ds(k * LANES, LANES)] for k in range(EMBED_DIM // LANES)]
    bias = bias_v[:]

    bufs = (tv0, tv1)
    sems = (sem0, sem1)

    def start(c):
        off = pl.multiple_of(base_v + c * SC_MV_CHUNK, 128)
        return pltpu.async_copy(
            tp_hbm.at[:, pl.ds(off, SC_MV_CHUNK)], bufs[c % 2], sems[c % 2]
        )

    pending = start(0)
    for c in range(SC_MV_STEPS):
        nxt = start(c + 1) if c + 1 < SC_MV_STEPS else None
        pending.wait()
        tv = bufs[c % 2]

        def grp(l, _):
            off = l * LANES
            acc = bias
            for d in range(EMBED_DIM):
                acc = acc + tv[d, pl.ds(off, LANES)] * w_vecs[d // LANES][d % LANES]
            out_v[pl.ds(c * SC_MV_CHUNK + off, LANES)] = acc
            return 0

        lax.fori_loop(0, SC_MV_CHUNK // LANES, grp, 0)
        pending = nxt

    pltpu.sync_copy(out_v, out_hbm.at[pl.ds(base_v, SC_PER_W)])


def _table_dot_w_sc(tphys, w_flat, bias_vec):
    mesh = plsc.VectorSubcoreMesh(core_axis_name="c", subcore_axis_name="s")
    kern = pl.kernel(
        _sc_mv_body,
        out_type=jax.ShapeDtypeStruct((SC_V,), jnp.float32),
        mesh=mesh,
        compiler_params=pltpu.CompilerParams(needs_layout_passes=False),
        scratch_types=[
            pltpu.VMEM((EMBED_DIM, SC_MV_CHUNK), jnp.float32),
            pltpu.VMEM((EMBED_DIM, SC_MV_CHUNK), jnp.float32),
            pltpu.VMEM((EMBED_DIM,), jnp.float32),
            pltpu.VMEM((LANES,), jnp.float32),
            pltpu.VMEM((SC_PER_W,), jnp.float32),
            pltpu.SemaphoreType.DMA,
            pltpu.SemaphoreType.DMA,
        ],
    )
    return kern(tphys, w_flat, bias_vec)


def _sc_gather_body(p_hbm, x_hbm, out_hbm, idx_v, rows_v, lanes_v, vals_v,
                    out_v, sem):
    wid = lax.axis_index("s") * 2 + lax.axis_index("c")
    base = wid * B_PER_W

    for j in range(N_CHUNKS):
        pltpu.sync_copy(x_hbm.at[pl.ds(base + j * GATHER_CHUNK, GATHER_CHUNK)],
                        idx_v.at[j])

    for j in range(N_CHUNKS):
        for k in range(GATHER_CHUNK // LANES):
            s = idx_v[j, pl.ds(k * LANES, LANES)]
            rows_v[j, pl.ds(k * LANES, LANES)] = s >> 4
            lanes_v[pl.ds(j * GATHER_CHUNK + k * LANES, LANES)] = s & 15

    copies = [
        pltpu.async_copy(
            p_hbm.at[rows_v.at[j]],
            vals_v.at[pl.ds(j * GATHER_CHUNK, GATHER_CHUNK)],
            sem,
        )
        for j in range(N_CHUNKS)
    ]
    for c in copies:
        c.wait()

    def group_body(g, _):
        b_ids = g * LANES + lax.iota(jnp.int32, LANES)
        lanes = lanes_v[pl.ds(g * LANES, LANES)]
        out_v[pl.ds(g * LANES, LANES)] = plsc.load_gather(vals_v, [b_ids, lanes])
        return 0

    lax.fori_loop(0, GROUPS, group_body, 0)

    pltpu.sync_copy(out_v, out_hbm.at[pl.ds(base, B_PER_W)])


def _sc_gather(p16, x):
    mesh = plsc.VectorSubcoreMesh(core_axis_name="c", subcore_axis_name="s")
    kern = pl.kernel(
        _sc_gather_body,
        out_type=jax.ShapeDtypeStruct((BATCH,), jnp.float32),
        mesh=mesh,
        compiler_params=pltpu.CompilerParams(
            needs_layout_passes=False, use_tc_tiling_on_sc=False
        ),
        scratch_types=[
            pltpu.VMEM((N_CHUNKS, GATHER_CHUNK), jnp.int32),
            pltpu.VMEM((N_CHUNKS, GATHER_CHUNK), jnp.int32),
            pltpu.VMEM((B_PER_W,), jnp.int32),
            pltpu.VMEM((B_PER_W, LANES), jnp.float32),
            pltpu.VMEM((B_PER_W,), jnp.float32),
            pltpu.SemaphoreType.DMA,
        ],
    )
    return kern(p16, x)


@jax.jit
def _run(x, table, W, b):
    tphys = table.T
    w = W.reshape(EMBED_DIM, 1)
    b11 = b.reshape(1, 1).astype(jnp.float32)
    w_flat = W.reshape(EMBED_DIM)
    bias_vec = jnp.broadcast_to(b.astype(jnp.float32), (LANES,))
    p_sc = _table_dot_w_sc(tphys, w_flat, bias_vec)
    p_tc = _table_dot_w_tc(tphys, w, b11)
    p = jnp.concatenate([p_sc, p_tc])
    p16 = p.reshape(P_ROWS, LANES)
    idx = x.astype(jnp.int32)
    return _sc_gather(p16, idx)


def kernel(x, table, W, b):
    return _run(x, table, W, b).reshape(BATCH, 1)


# speedup vs baseline: 1.0032x; 1.0032x over previous
import jax
import jax.numpy as jnp
from jax import lax
from jax.experimental import pallas as pl
from jax.experimental.pallas import tpu as pltpu
from jax.experimental.pallas import tpu_sc as plsc

VOCAB = 1000000
EMBED_DIM = 64
BATCH = 16384
LANES = 16
NUM_WORKERS = 32
B_PER_W = BATCH // NUM_WORKERS
GATHER_CHUNK = 128
N_CHUNKS = B_PER_W // GATHER_CHUNK
GROUPS = B_PER_W // LANES
P_ROWS = VOCAB // LANES

TC_CHUNK = 32768
SC_V = 262144
SC_PER_W = SC_V // NUM_WORKERS
SC_MV_CHUNK = 512
SC_MV_STEPS = SC_PER_W // SC_MV_CHUNK
TC_V = VOCAB - SC_V
TC_GRID = (TC_V + TC_CHUNK - 1) // TC_CHUNK
TC_BLK0 = SC_V // TC_CHUNK


def _matvec_body(t_ref, w_ref, b_ref, p_ref):
    p_ref[...] = jnp.sum(t_ref[...] * w_ref[...], axis=0) + b_ref[0, 0]


def _table_dot_w_tc(tphys, w, b11):
    return pl.pallas_call(
        _matvec_body,
        grid=(TC_GRID,),
        in_specs=[
            pl.BlockSpec((EMBED_DIM, TC_CHUNK), lambda i: (0, i + TC_BLK0)),
            pl.BlockSpec((EMBED_DIM, 1), lambda i: (0, 0)),
            pl.BlockSpec((1, 1), lambda i: (0, 0)),
        ],
        out_specs=pl.BlockSpec((TC_CHUNK,), lambda i: (i,)),
        out_shape=jax.ShapeDtypeStruct((TC_V,), jnp.float32),
    )(tphys, w, b11)


def _sc_mv_body(tp_hbm, w_hbm, bias_hbm, out_hbm, tv0, tv1, w_v, bias_v,
                out_v, sem0, sem1):
    wid = lax.axis_index("s") * 2 + lax.axis_index("c")
    base_v = pl.multiple_of(wid * SC_PER_W, 1024)

    pltpu.sync_copy(w_hbm, w_v)
    pltpu.sync_copy(bias_hbm, bias_v)
    w_vecs = [w_v[pl.ds(k * LANES, LANES)] for k in range(EMBED_DIM // LANES)]
    bias = bias_v[:]

    bufs = (tv0, tv1)
    sems = (sem0, sem1)

    def start(c, buf_i):
        off = pl.multiple_of(base_v + c * SC_MV_CHUNK, 128)
        return pltpu.async_copy(
            tp_hbm.at[:, pl.ds(off, SC_MV_CHUNK)], bufs[buf_i], sems[buf_i]
        )

    start(0, 0)

    def pair(cc, _):
        for b in range(2):
            c = cc * 2 + b

            @pl.when(c + 1 < SC_MV_STEPS)
            def _():
                start(c + 1, (b + 1) % 2)

            pltpu.make_async_copy(
                tp_hbm.at[:, pl.ds(pl.multiple_of(base_v, 128), SC_MV_CHUNK)],
                bufs[b], sems[b],
            ).wait()
            tv = bufs[b]

            def grp(l, _, tv=tv, c=c):
                off = l * LANES
                acc = bias
                for d in range(EMBED_DIM):
                    acc = acc + tv[d, pl.ds(off, LANES)] * w_vecs[d // LANES][d % LANES]
                out_v[pl.ds(c * SC_MV_CHUNK + off, LANES)] = acc
                return 0

            lax.fori_loop(0, SC_MV_CHUNK // LANES, grp, 0, unroll=4)
        return 0

    lax.fori_loop(0, SC_MV_STEPS // 2, pair, 0)

    pltpu.sync_copy(out_v, out_hbm.at[pl.ds(base_v, SC_PER_W)])


def _table_dot_w_sc(tphys, w_flat, bias_vec):
    mesh = plsc.VectorSubcoreMesh(core_axis_name="c", subcore_axis_name="s")
    kern = pl.kernel(
        _sc_mv_body,
        out_type=jax.ShapeDtypeStruct((SC_V,), jnp.float32),
        mesh=mesh,
        compiler_params=pltpu.CompilerParams(needs_layout_passes=False),
        scratch_types=[
            pltpu.VMEM((EMBED_DIM, SC_MV_CHUNK), jnp.float32),
            pltpu.VMEM((EMBED_DIM, SC_MV_CHUNK), jnp.float32),
            pltpu.VMEM((EMBED_DIM,), jnp.float32),
            pltpu.VMEM((LANES,), jnp.float32),
            pltpu.VMEM((SC_PER_W,), jnp.float32),
            pltpu.SemaphoreType.DMA,
            pltpu.SemaphoreType.DMA,
        ],
    )
    return kern(tphys, w_flat, bias_vec)


def _sc_gather_body(p_hbm, x_hbm, out_hbm, idx_v, rows_v, lanes_v, vals_v,
                    out_v, sem):
    wid = lax.axis_index("s") * 2 + lax.axis_index("c")
    base = wid * B_PER_W

    for j in range(N_CHUNKS):
        pltpu.sync_copy(x_hbm.at[pl.ds(base + j * GATHER_CHUNK, GATHER_CHUNK)],
                        idx_v.at[j])

    for j in range(N_CHUNKS):
        for k in range(GATHER_CHUNK // LANES):
            s = idx_v[j, pl.ds(k * LANES, LANES)]
            rows_v[j, pl.ds(k * LANES, LANES)] = s >> 4
            lanes_v[pl.ds(j * GATHER_CHUNK + k * LANES, LANES)] = s & 15

    copies = [
        pltpu.async_copy(
            p_hbm.at[rows_v.at[j]],
            vals_v.at[pl.ds(j * GATHER_CHUNK, GATHER_CHUNK)],
            sem,
        )
        for j in range(N_CHUNKS)
    ]
    for c in copies:
        c.wait()

    def group_body(g, _):
        b_ids = g * LANES + lax.iota(jnp.int32, LANES)
        lanes = lanes_v[pl.ds(g * LANES, LANES)]
        out_v[pl.ds(g * LANES, LANES)] = plsc.load_gather(vals_v, [b_ids, lanes])
        return 0

    lax.fori_loop(0, GROUPS, group_body, 0)

    pltpu.sync_copy(out_v, out_hbm.at[pl.ds(base, B_PER_W)])


def _sc_gather(p16, x):
    mesh = plsc.VectorSubcoreMesh(core_axis_name="c", subcore_axis_name="s")
    kern = pl.kernel(
        _sc_gather_body,
        out_type=jax.ShapeDtypeStruct((BATCH,), jnp.float32),
        mesh=mesh,
        compiler_params=pltpu.CompilerParams(
            needs_layout_passes=False, use_tc_tiling_on_sc=False
        ),
        scratch_types=[
            pltpu.VMEM((N_CHUNKS, GATHER_CHUNK), jnp.int32),
            pltpu.VMEM((N_CHUNKS, GATHER_CHUNK), jnp.int32),
            pltpu.VMEM((B_PER_W,), jnp.int32),
            pltpu.VMEM((B_PER_W, LANES), jnp.float32),
            pltpu.VMEM((B_PER_W,), jnp.float32),
            pltpu.SemaphoreType.DMA,
        ],
    )
    return kern(p16, x)


@jax.jit
def _run(x, table, W, b):
    tphys = table.T
    w = W.reshape(EMBED_DIM, 1)
    b11 = b.reshape(1, 1).astype(jnp.float32)
    w_flat = W.reshape(EMBED_DIM)
    bias_vec = jnp.broadcast_to(b.astype(jnp.float32), (LANES,))
    p_sc = _table_dot_w_sc(tphys, w_flat, bias_vec)
    p_tc = _table_dot_w_tc(tphys, w, b11)
    p = jnp.concatenate([p_sc, p_tc])
    p16 = p.reshape(P_ROWS, LANES)
    idx = x.astype(jnp.int32)
    return _sc_gather(p16, idx)


def kernel(x, table, W, b):
    return _run(x, table, W, b).reshape(BATCH, 1)


# speedup vs baseline: 1.0588x; 1.0555x over previous
import jax
import jax.numpy as jnp
from jax import lax
from jax.experimental import pallas as pl
from jax.experimental.pallas import tpu as pltpu
from jax.experimental.pallas import tpu_sc as plsc

VOCAB = 1000000
EMBED_DIM = 64
BATCH = 16384
LANES = 16
NUM_WORKERS = 32
B_PER_W = BATCH // NUM_WORKERS
GATHER_CHUNK = 128
N_CHUNKS = B_PER_W // GATHER_CHUNK
GROUPS = B_PER_W // LANES
P_ROWS = VOCAB // LANES

TC_CHUNK = 32768
TC_GRID = (VOCAB + TC_CHUNK - 1) // TC_CHUNK


def _matvec_body(t_ref, w_ref, b_ref, p_ref):
    p_ref[...] = jnp.sum(t_ref[...] * w_ref[...], axis=0) + b_ref[0, 0]


def _table_dot_w(tphys, w, b11):
    return pl.pallas_call(
        _matvec_body,
        grid=(TC_GRID,),
        in_specs=[
            pl.BlockSpec((EMBED_DIM, TC_CHUNK), lambda i: (0, i)),
            pl.BlockSpec((EMBED_DIM, 1), lambda i: (0, 0)),
            pl.BlockSpec((1, 1), lambda i: (0, 0)),
        ],
        out_specs=pl.BlockSpec((TC_CHUNK,), lambda i: (i,)),
        out_shape=jax.ShapeDtypeStruct((VOCAB,), jnp.float32),
    )(tphys, w, b11)


def _sc_gather_body(p_hbm, x_hbm, out_hbm, idx_v, rows_v, lanes_v, vals_v,
                    out_v, sem):
    wid = lax.axis_index("s") * 2 + lax.axis_index("c")
    base = wid * B_PER_W

    for j in range(N_CHUNKS):
        pltpu.sync_copy(x_hbm.at[pl.ds(base + j * GATHER_CHUNK, GATHER_CHUNK)],
                        idx_v.at[j])

    for j in range(N_CHUNKS):
        for k in range(GATHER_CHUNK // LANES):
            s = idx_v[j, pl.ds(k * LANES, LANES)]
            rows_v[j, pl.ds(k * LANES, LANES)] = s >> 4
            lanes_v[pl.ds(j * GATHER_CHUNK + k * LANES, LANES)] = s & 15

    copies = [
        pltpu.async_copy(
            p_hbm.at[rows_v.at[j]],
            vals_v.at[pl.ds(j * GATHER_CHUNK, GATHER_CHUNK)],
            sem,
        )
        for j in range(N_CHUNKS)
    ]
    for c in copies:
        c.wait()

    def group_body(g, _):
        b_ids = g * LANES + lax.iota(jnp.int32, LANES)
        lanes = lanes_v[pl.ds(g * LANES, LANES)]
        out_v[pl.ds(g * LANES, LANES)] = plsc.load_gather(vals_v, [b_ids, lanes])
        return 0

    lax.fori_loop(0, GROUPS, group_body, 0)

    pltpu.sync_copy(out_v, out_hbm.at[pl.ds(base, B_PER_W)])


def _sc_gather(p16, x):
    mesh = plsc.VectorSubcoreMesh(core_axis_name="c", subcore_axis_name="s")
    kern = pl.kernel(
        _sc_gather_body,
        out_type=jax.ShapeDtypeStruct((BATCH,), jnp.float32),
        mesh=mesh,
        compiler_params=pltpu.CompilerParams(
            needs_layout_passes=False, use_tc_tiling_on_sc=False
        ),
        scratch_types=[
            pltpu.VMEM((N_CHUNKS, GATHER_CHUNK), jnp.int32),
            pltpu.VMEM((N_CHUNKS, GATHER_CHUNK), jnp.int32),
            pltpu.VMEM((B_PER_W,), jnp.int32),
            pltpu.VMEM((B_PER_W, LANES), jnp.float32),
            pltpu.VMEM((B_PER_W,), jnp.float32),
            pltpu.SemaphoreType.DMA,
        ],
    )
    return kern(p16, x)


@jax.jit
def _run(x, table, W, b):
    tphys = table.T
    w = W.reshape(EMBED_DIM, 1)
    b11 = b.reshape(1, 1).astype(jnp.float32)
    p = _table_dot_w(tphys, w, b11)
    p16 = p.reshape(P_ROWS, LANES)
    idx = x.astype(jnp.int32)
    return _sc_gather(p16, idx)


def kernel(x, table, W, b):
    return _run(x, table, W, b).reshape(BATCH, 1)
